# Initial kernel scaffold; baseline (speedup 1.0000x reference)
import jax
import jax.numpy as jnp
from jax.experimental import pallas as pl


def kernel(users, items, edge_index, user_table, item_table, edge_weight):
    raise NotImplementedError("write your pallas kernel here")


# speedup vs baseline: 13.6029x; 13.6029x over previous
import functools

import jax
import jax.numpy as jnp
from jax import lax
from jax.experimental import pallas as pl
from jax.experimental.pallas import tpu as pltpu
import jax.experimental.pallas.tpu_sc as plsc

NU = 50000
NI = 50000
NN = NU + NI
HALF = NN // 2
E = 1600000
D = 32
B = 16384

NC = 2
NS = 16

W = 800
SUB = 80
NSUB = W // SUB
CH = E // NS
NW = CH // W
FCH = 400
NFC = HALF // FCH

_mesh = plsc.VectorSubcoreMesh(core_axis_name="c", subcore_axis_name="s")


@functools.partial(
    pl.kernel,
    out_type=jax.ShapeDtypeStruct((NN, D), jnp.float32),
    mesh=_mesh,
    compiler_params=pltpu.CompilerParams(use_tc_tiling_on_sc=False, needs_layout_passes=False),
    scratch_types=[
        pltpu.VMEM((W,), jnp.int32),
        pltpu.VMEM((W,), jnp.int32),
        pltpu.VMEM((W,), jnp.float32),
        pltpu.VMEM((W, D), jnp.float32),
        pltpu.VMEM_SHARED((HALF, D), jnp.float32),
        pltpu.SemaphoreType.DMA,
        pltpu.SemaphoreType.DMA,
    ],
)
def _layer(emb, src1, dst1, w1, out, src_v, dst_v, w_v, rows_v, acc, gsem, ssem):
    c = lax.axis_index("c")
    s = lax.axis_index("s")
    lo = c * HALF
    iota = lax.iota(jnp.int32, 16)

    def _zrow(r, _):
        rows_v[r, pl.ds(0, 16)] = jnp.zeros((16,), jnp.float32)
        rows_v[r, pl.ds(16, 16)] = jnp.zeros((16,), jnp.float32)
        return 0

    lax.fori_loop(0, FCH, _zrow, 0)
    nch = lax.select(s < NFC % NS, NFC // NS + 1, NFC // NS)

    def _zchunk(k, _):
        ch = s + k * NS
        pltpu.sync_copy(rows_v.at[pl.ds(0, FCH)], acc.at[pl.ds(ch * FCH, FCH)])
        return 0

    lax.fori_loop(0, nch, _zchunk, 0)
    plsc.subcore_barrier()

    def _window(widx, _):
        base = s * CH + widx * W
        pltpu.sync_copy(src1.at[pl.ds(base, W)], src_v)
        for j in range(NSUB):
            pltpu.async_copy(
                emb.at[src_v.at[pl.ds(j * SUB, SUB)]],
                rows_v.at[pl.ds(j * SUB, SUB)],
                gsem,
            )
        pltpu.sync_copy(dst1.at[pl.ds(base, W)], dst_v)
        pltpu.sync_copy(w1.at[pl.ds(base, W)], w_v)
        for v in range(W // 16):
            dv = dst_v[pl.ds(v * 16, 16)]
            wv = w_v[pl.ds(v * 16, 16)]
            m = (dv >= lo) & (dv < lo + HALF)
            dst_v[pl.ds(v * 16, 16)] = jnp.where(m, dv - lo, iota + v * 16)
            w_v[pl.ds(v * 16, 16)] = jnp.where(m, wv, 0.0)
        for j in range(NSUB):
            pltpu.make_async_copy(
                emb.at[src_v.at[pl.ds(j * SUB, SUB)]],
                rows_v.at[pl.ds(j * SUB, SUB)],
                gsem,
            ).wait()
        for j in range(NSUB):
            def _scale(cc, _):
                off = j * SUB + cc * 16
                wv = w_v[pl.ds(off, 16)]
                for k in range(16):
                    e = off + k
                    ws = wv[k]
                    rows_v[e, pl.ds(0, 16)] = rows_v[e, pl.ds(0, 16)] * ws
                    rows_v[e, pl.ds(16, 16)] = rows_v[e, pl.ds(16, 16)] * ws
                return 0

            lax.fori_loop(0, SUB // 16, _scale, 0)
        for j in range(NSUB):
            pltpu.async_copy(
                rows_v.at[pl.ds(j * SUB, SUB)],
                acc.at[dst_v.at[pl.ds(j * SUB, SUB)]],
                ssem,
                add=True,
            )
        for j in range(NSUB):
            pltpu.make_async_copy(
                rows_v.at[pl.ds(j * SUB, SUB)],
                acc.at[dst_v.at[pl.ds(j * SUB, SUB)]],
                ssem,
            ).wait()
        return 0

    lax.fori_loop(0, NW, _window, 0)
    plsc.subcore_barrier()

    def _fchunk(k, _):
        ch = s + k * NS
        pltpu.sync_copy(
            acc.at[pl.ds(ch * FCH, FCH)],
            out.at[pl.ds(lo + ch * FCH, FCH)],
        )
        return 0

    lax.fori_loop(0, nch, _fchunk, 0)


BSUB = 128
BPER = B // (NC * NS)
BNS = BPER // BSUB


@functools.partial(
    pl.kernel,
    out_type=(
        jax.ShapeDtypeStruct((B, D), jnp.float32),
        jax.ShapeDtypeStruct((B, D), jnp.float32),
        jax.ShapeDtypeStruct((B,), jnp.float32),
    ),
    mesh=_mesh,
    compiler_params=pltpu.CompilerParams(use_tc_tiling_on_sc=False, needs_layout_passes=False),
    scratch_types=[
        pltpu.VMEM((BPER,), jnp.int32),
        pltpu.VMEM((BPER, D), jnp.float32),
        pltpu.VMEM((BPER, D), jnp.float32),
        pltpu.VMEM((BPER, D), jnp.float32),
        pltpu.VMEM((BPER, D), jnp.float32),
        pltpu.VMEM((BPER,), jnp.float32),
        pltpu.SemaphoreType.DMA,
    ],
)
def _final(e0, e1, e2, users1, items1, ue, ie, sc_out,
           idx_v, r0, r1, r2, um, sv, sem):
    c = lax.axis_index("c")
    s = lax.axis_index("s")
    wid = s * NC + c
    base = wid * BPER
    third = jnp.float32(1.0 / 3.0)

    def _gather3():
        for j in range(BNS):
            ii = idx_v.at[pl.ds(j * BSUB, BSUB)]
            pltpu.async_copy(e0.at[ii], r0.at[pl.ds(j * BSUB, BSUB)], sem)
            pltpu.async_copy(e1.at[ii], r1.at[pl.ds(j * BSUB, BSUB)], sem)
            pltpu.async_copy(e2.at[ii], r2.at[pl.ds(j * BSUB, BSUB)], sem)
        for j in range(BNS):
            ii = idx_v.at[pl.ds(j * BSUB, BSUB)]
            for rr, dd in ((r0, e0), (r1, e1), (r2, e2)):
                pltpu.make_async_copy(
                    dd.at[ii], rr.at[pl.ds(j * BSUB, BSUB)], sem
                ).wait()

    pltpu.sync_copy(users1.at[pl.ds(base, BPER)], idx_v)
    _gather3()

    def _umean(e, _):
        for h in range(2):
            um[e, pl.ds(h * 16, 16)] = (
                r0[e, pl.ds(h * 16, 16)]
                + r1[e, pl.ds(h * 16, 16)]
                + r2[e, pl.ds(h * 16, 16)]
            ) * third
        return 0

    lax.fori_loop(0, BPER, _umean, 0)
    pltpu.sync_copy(um, ue.at[pl.ds(base, BPER)])

    pltpu.sync_copy(items1.at[pl.ds(base, BPER)], idx_v)
    for v in range(BPER // 16):
        idx_v[pl.ds(v * 16, 16)] = idx_v[pl.ds(v * 16, 16)] + NU
    _gather3()

    iota = lax.iota(jnp.int32, 16)

    def _imean(g, _):
        svec = jnp.zeros((16,), jnp.float32)
        for k in range(16):
            e = g * 16 + k
            acc0 = jnp.zeros((16,), jnp.float32)
            for h in range(2):
                im = (
                    r0[e, pl.ds(h * 16, 16)]
                    + r1[e, pl.ds(h * 16, 16)]
                    + r2[e, pl.ds(h * 16, 16)]
                ) * third
                r0[e, pl.ds(h * 16, 16)] = im
                acc0 = acc0 + im * um[e, pl.ds(h * 16, 16)]
            svec = jnp.where(iota == k, jnp.sum(acc0), svec)
        sv[pl.ds(g * 16, 16)] = svec
        return 0

    lax.fori_loop(0, BPER // 16, _imean, 0)
    pltpu.sync_copy(r0, ie.at[pl.ds(base, BPER)])
    pltpu.sync_copy(sv, sc_out.at[pl.ds(base, BPER)])


@jax.jit
def kernel(users, items, edge_index, user_table, item_table, edge_weight):
    emb0 = jnp.concatenate([user_table, item_table], axis=0)
    src1 = edge_index[0]
    dst1 = edge_index[1]
    emb1 = _layer(emb0, src1, dst1, edge_weight)
    emb2 = _layer(emb1, src1, dst1, edge_weight)
    return _final(emb0, emb1, emb2, users, items)


# speedup vs baseline: 21.4556x; 1.5773x over previous
import functools

import jax
import jax.numpy as jnp
from jax import lax
from jax.experimental import pallas as pl
from jax.experimental.pallas import tpu as pltpu
import jax.experimental.pallas.tpu_sc as plsc

NU = 50000
NI = 50000
NN = NU + NI
HALF = NN // 2
E = 1600000
D = 32
B = 16384

NC = 2
NS = 16

W = 400
SUB = 400
NSUB = W // SUB
CH = E // NS
NW = CH // W
FCH = 400
NFC = HALF // FCH

_mesh = plsc.VectorSubcoreMesh(core_axis_name="c", subcore_axis_name="s")


@functools.partial(
    pl.kernel,
    out_type=jax.ShapeDtypeStruct((NN, D), jnp.float32),
    mesh=_mesh,
    compiler_params=pltpu.CompilerParams(
        use_tc_tiling_on_sc=False, needs_layout_passes=False
    ),
    scratch_types=[
        [pltpu.VMEM((W,), jnp.int32)] * 2,
        [pltpu.VMEM((W,), jnp.int32)] * 2,
        [pltpu.VMEM((W,), jnp.float32)] * 2,
        [pltpu.VMEM((W,), jnp.int32)] * 2,
        [pltpu.VMEM((W,), jnp.float32)] * 2,
        [pltpu.VMEM((W, D), jnp.float32)] * 2,
        pltpu.VMEM_SHARED((HALF, D), jnp.float32),
        [pltpu.SemaphoreType.DMA] * 2,
        [pltpu.SemaphoreType.DMA] * 2,
        [pltpu.SemaphoreType.DMA] * 2,
    ],
)
def _layer(emb, src1, dst1, w1, out, rsrc, rdst, rw, cdst, cw, rows_v,
           acc, isem, gsem, ssem):
    c = lax.axis_index("c")
    s = lax.axis_index("s")
    lo = c * HALF
    iota = lax.iota(jnp.int32, 16)
    zf = jnp.zeros((16,), jnp.float32)

    def _zrow(r, _):
        rows_v[0][r, pl.ds(0, 16)] = zf
        rows_v[0][r, pl.ds(16, 16)] = zf
        return 0

    lax.fori_loop(0, FCH, _zrow, 0)
    nch = lax.select(s < NFC % NS, NFC // NS + 1, NFC // NS)

    def _zchunk(k, _):
        ch = s + k * NS
        pltpu.sync_copy(
            rows_v[0].at[pl.ds(0, FCH)], acc.at[pl.ds(ch * FCH, FCH)]
        )
        return 0

    lax.fori_loop(0, nch, _zchunk, 0)
    plsc.subcore_barrier()

    def _fire_idx(b, widx):
        base = s * CH + widx * W
        pltpu.async_copy(src1.at[pl.ds(base, W)], rsrc[b], isem[b])
        pltpu.async_copy(dst1.at[pl.ds(base, W)], rdst[b], isem[b])
        pltpu.async_copy(w1.at[pl.ds(base, W)], rw[b], isem[b])

    def _wait_idx(b, widx):
        base = s * CH + widx * W
        pltpu.make_async_copy(src1.at[pl.ds(base, W)], rsrc[b], isem[b]).wait()
        pltpu.make_async_copy(dst1.at[pl.ds(base, W)], rdst[b], isem[b]).wait()
        pltpu.make_async_copy(w1.at[pl.ds(base, W)], rw[b], isem[b]).wait()

    def _fire_gather(b):
        for j in range(NSUB):
            pltpu.async_copy(
                emb.at[rsrc[b].at[pl.ds(j * SUB, SUB)]],
                rows_v[b].at[pl.ds(j * SUB, SUB)],
                gsem[b],
            )

    def _drain_gather(b):
        for j in range(NSUB):
            pltpu.make_async_copy(
                emb.at[rsrc[b].at[pl.ds(j * SUB, SUB)]],
                rows_v[b].at[pl.ds(j * SUB, SUB)],
                gsem[b],
            ).wait()

    def _mask(b):
        for v in range(W // 16):
            dv = rdst[b][pl.ds(v * 16, 16)]
            wv = rw[b][pl.ds(v * 16, 16)]
            m = (dv >= lo) & (dv < lo + HALF)
            cdst[b][pl.ds(v * 16, 16)] = jnp.where(m, dv - lo, iota + v * 16)
            cw[b][pl.ds(v * 16, 16)] = jnp.where(m, wv, 0.0)

    def _scale(b):
        for j in range(NSUB):
            def _sc(cc, _):
                off = j * SUB + cc * 16
                wv = cw[b][pl.ds(off, 16)]
                for k in range(16):
                    e = off + k
                    ws = wv[k]
                    rows_v[b][e, pl.ds(0, 16)] = rows_v[b][e, pl.ds(0, 16)] * ws
                    rows_v[b][e, pl.ds(16, 16)] = (
                        rows_v[b][e, pl.ds(16, 16)] * ws
                    )
                return 0

            lax.fori_loop(0, SUB // 16, _sc, 0)

    def _fire_scatter(b):
        for j in range(NSUB):
            pltpu.async_copy(
                rows_v[b].at[pl.ds(j * SUB, SUB)],
                acc.at[cdst[b].at[pl.ds(j * SUB, SUB)]],
                ssem[b],
                add=True,
            )

    def _drain_scatter(b):
        for j in range(NSUB):
            pltpu.make_async_copy(
                rows_v[b].at[pl.ds(j * SUB, SUB)],
                acc.at[cdst[b].at[pl.ds(j * SUB, SUB)]],
                ssem[b],
            ).wait()

    def _half(b, nb, gnext, gpre, first=False, last=0):
        if not first:
            _drain_scatter(nb)
        if gnext is not None:
            _wait_idx(nb, gnext)
            _fire_gather(nb)
        _drain_gather(b)
        _mask(b)
        if gpre is not None:
            _fire_idx(b, gpre)
        _scale(b)
        _fire_scatter(b)

    _fire_idx(0, 0)
    _fire_idx(1, 1)
    _wait_idx(0, 0)
    _fire_gather(0)
    _half(0, 1, 1, 2, first=True)

    def _pair(t, _):
        g = 2 * t + 1
        _half(1, 0, g + 1, g + 2)
        _half(0, 1, g + 2, g + 3)
        return 0

    lax.fori_loop(0, (NW - 4) // 2, _pair, 0)

    _half(1, 0, NW - 2, NW - 1)
    _half(0, 1, NW - 1, None)
    _half(1, 0, None, None)
    _drain_scatter(1)
    plsc.subcore_barrier()

    def _fchunk(k, _):
        ch = s + k * NS
        pltpu.sync_copy(
            acc.at[pl.ds(ch * FCH, FCH)],
            out.at[pl.ds(lo + ch * FCH, FCH)],
        )
        return 0

    lax.fori_loop(0, nch, _fchunk, 0)


BSUB = 128
BPER = B // (NC * NS)
BNS = BPER // BSUB


@functools.partial(
    pl.kernel,
    out_type=(
        jax.ShapeDtypeStruct((B, D), jnp.float32),
        jax.ShapeDtypeStruct((B, D), jnp.float32),
        jax.ShapeDtypeStruct((B,), jnp.float32),
    ),
    mesh=_mesh,
    compiler_params=pltpu.CompilerParams(use_tc_tiling_on_sc=False, needs_layout_passes=False),
    scratch_types=[
        pltpu.VMEM((BPER,), jnp.int32),
        pltpu.VMEM((BPER, D), jnp.float32),
        pltpu.VMEM((BPER, D), jnp.float32),
        pltpu.VMEM((BPER, D), jnp.float32),
        pltpu.VMEM((BPER, D), jnp.float32),
        pltpu.VMEM((BPER,), jnp.float32),
        pltpu.SemaphoreType.DMA,
    ],
)
def _final(e0, e1, e2, users1, items1, ue, ie, sc_out,
           idx_v, r0, r1, r2, um, sv, sem):
    c = lax.axis_index("c")
    s = lax.axis_index("s")
    wid = s * NC + c
    base = wid * BPER
    third = jnp.float32(1.0 / 3.0)

    def _gather3():
        for j in range(BNS):
            ii = idx_v.at[pl.ds(j * BSUB, BSUB)]
            pltpu.async_copy(e0.at[ii], r0.at[pl.ds(j * BSUB, BSUB)], sem)
            pltpu.async_copy(e1.at[ii], r1.at[pl.ds(j * BSUB, BSUB)], sem)
            pltpu.async_copy(e2.at[ii], r2.at[pl.ds(j * BSUB, BSUB)], sem)
        for j in range(BNS):
            ii = idx_v.at[pl.ds(j * BSUB, BSUB)]
            for rr, dd in ((r0, e0), (r1, e1), (r2, e2)):
                pltpu.make_async_copy(
                    dd.at[ii], rr.at[pl.ds(j * BSUB, BSUB)], sem
                ).wait()

    pltpu.sync_copy(users1.at[pl.ds(base, BPER)], idx_v)
    _gather3()

    def _umean(e, _):
        for h in range(2):
            um[e, pl.ds(h * 16, 16)] = (
                r0[e, pl.ds(h * 16, 16)]
                + r1[e, pl.ds(h * 16, 16)]
                + r2[e, pl.ds(h * 16, 16)]
            ) * third
        return 0

    lax.fori_loop(0, BPER, _umean, 0)
    pltpu.sync_copy(um, ue.at[pl.ds(base, BPER)])

    pltpu.sync_copy(items1.at[pl.ds(base, BPER)], idx_v)
    for v in range(BPER // 16):
        idx_v[pl.ds(v * 16, 16)] = idx_v[pl.ds(v * 16, 16)] + NU
    _gather3()

    iota = lax.iota(jnp.int32, 16)

    def _imean(g, _):
        svec = jnp.zeros((16,), jnp.float32)
        for k in range(16):
            e = g * 16 + k
            acc0 = jnp.zeros((16,), jnp.float32)
            for h in range(2):
                im = (
                    r0[e, pl.ds(h * 16, 16)]
                    + r1[e, pl.ds(h * 16, 16)]
                    + r2[e, pl.ds(h * 16, 16)]
                ) * third
                r0[e, pl.ds(h * 16, 16)] = im
                acc0 = acc0 + im * um[e, pl.ds(h * 16, 16)]
            svec = jnp.where(iota == k, jnp.sum(acc0), svec)
        sv[pl.ds(g * 16, 16)] = svec
        return 0

    lax.fori_loop(0, BPER // 16, _imean, 0)
    pltpu.sync_copy(r0, ie.at[pl.ds(base, BPER)])
    pltpu.sync_copy(sv, sc_out.at[pl.ds(base, BPER)])


@jax.jit
def kernel(users, items, edge_index, user_table, item_table, edge_weight):
    emb0 = jnp.concatenate([user_table, item_table], axis=0)
    src1 = edge_index[0]
    dst1 = edge_index[1]
    emb1 = _layer(emb0, src1, dst1, edge_weight)
    emb2 = _layer(emb1, src1, dst1, edge_weight)
    return _final(emb0, emb1, emb2, users, items)
